# baseline (speedup 1.0000x reference)
import jax
import jax.numpy as jnp
from jax import lax
from jax.experimental import pallas as pl
from jax.experimental.pallas import tpu as pltpu
from jax.experimental.pallas import tpu_sc as plsc

NUM_CORES = 2
NUM_SUBCORES = 16
NUM_WORKERS = NUM_CORES * NUM_SUBCORES

NUM_EMB = 1000000
B_TOTAL = 4096 * 200
D = 64
W = 2 * D
B_PER_W = B_TOTAL // NUM_WORKERS
CHUNK = 400
N_CHUNKS = B_PER_W // CHUNK


def _stage_in(x_hbm, table_hbm, base, g, idx_v, rows_v, sem):
    off = base + g * CHUNK
    pltpu.sync_copy(x_hbm.at[pl.ds(off, CHUNK)], idx_v)
    return pltpu.async_copy(table_hbm.at[idx_v], rows_v, sem)


def _drain(out_hbm, base, g, idx_v, rows_v, sem, table_hbm):
    off = base + g * CHUNK
    pltpu.make_async_copy(table_hbm.at[idx_v], rows_v, sem).wait()
    pltpu.sync_copy(rows_v, out_hbm.at[pl.ds(off, CHUNK)])


def _emb_body(x_hbm, table_hbm, out_hbm,
              idx_a, rows_a, sem_a, idx_b, rows_b, sem_b):
    wid = lax.axis_index("s") * NUM_CORES + lax.axis_index("c")
    base = wid * B_PER_W

    _stage_in(x_hbm, table_hbm, base, 0, idx_a, rows_a, sem_a)

    def body(t, carry):
        g0 = 2 * t
        _stage_in(x_hbm, table_hbm, base, g0 + 1, idx_b, rows_b, sem_b)
        _drain(out_hbm, base, g0, idx_a, rows_a, sem_a, table_hbm)

        @pl.when(t + 1 < N_CHUNKS // 2)
        def _():
            _stage_in(x_hbm, table_hbm, base, g0 + 2, idx_a, rows_a, sem_a)

        _drain(out_hbm, base, g0 + 1, idx_b, rows_b, sem_b, table_hbm)
        return carry

    lax.fori_loop(0, N_CHUNKS // 2, body, 0)


FMT_R = 512
FMT_GRID = (NUM_EMB + FMT_R - 1) // FMT_R


def _fmt_body(t_ref, out_ref):
    out_ref[:, :D] = t_ref[...].T
    out_ref[:, D:] = jnp.zeros((FMT_R, W - D), jnp.float32)


def _format_table(table):
    return pl.pallas_call(
        _fmt_body,
        out_shape=jax.ShapeDtypeStruct((NUM_EMB, W), jnp.float32),
        grid=(FMT_GRID,),
        in_specs=[pl.BlockSpec((D, FMT_R), lambda i: (0, i))],
        out_specs=pl.BlockSpec((FMT_R, W), lambda i: (i, 0)),
    )(table.T)


def _emb_lookup(x, table):
    mesh = plsc.VectorSubcoreMesh(core_axis_name="c", subcore_axis_name="s")
    x_flat = x.reshape(-1).astype(jnp.int32)
    table_pad = _format_table(table)
    out_p = pl.kernel(
        _emb_body,
        out_type=jax.ShapeDtypeStruct((B_TOTAL, W), jnp.float32),
        mesh=mesh,
        scratch_types=[
            pltpu.VMEM((CHUNK,), jnp.int32),
            pltpu.VMEM((CHUNK, W), jnp.float32),
            pltpu.SemaphoreType.DMA,
            pltpu.VMEM((CHUNK,), jnp.int32),
            pltpu.VMEM((CHUNK, W), jnp.float32),
            pltpu.SemaphoreType.DMA,
        ],
        compiler_params=pltpu.CompilerParams(use_tc_tiling_on_sc=False),
    )(x_flat, table_pad)
    return out_p.reshape(x.shape + (W,))[:, :, :D]


def kernel(x, embedding_weight):
    return _emb_lookup(x, embedding_weight)


# speedup vs baseline: 1.2410x; 1.2410x over previous
import jax
import jax.numpy as jnp
from jax import lax
from jax.experimental import pallas as pl
from jax.experimental.pallas import tpu as pltpu
from jax.experimental.pallas import tpu_sc as plsc

NUM_CORES = 2
NUM_SUBCORES = 16
NUM_WORKERS = NUM_CORES * NUM_SUBCORES

NUM_EMB = 1000000
B_TOTAL = 4096 * 200
D = 64
W = 2 * D
B_PER_W = B_TOTAL // NUM_WORKERS
CHUNK = 400
N_CHUNKS = B_PER_W // CHUNK


def _stage_in(x_hbm, table_hbm, base, g, idx_v, rows_v, sem):
    off = base + g * CHUNK
    pltpu.sync_copy(x_hbm.at[pl.ds(off, CHUNK)], idx_v)
    return pltpu.async_copy(table_hbm.at[idx_v], rows_v, sem)


def _drain(out_hbm, base, g, idx_v, rows_v, sem, table_hbm):
    off = base + g * CHUNK
    pltpu.make_async_copy(table_hbm.at[idx_v], rows_v, sem).wait()
    pltpu.sync_copy(rows_v, out_hbm.at[pl.ds(off, CHUNK)])


def _emb_body(x_hbm, table_hbm, out_hbm,
              idx_a, rows_a, sem_a, idx_b, rows_b, sem_b):
    wid = lax.axis_index("s") * NUM_CORES + lax.axis_index("c")
    base = wid * B_PER_W

    _stage_in(x_hbm, table_hbm, base, 0, idx_a, rows_a, sem_a)

    def body(t, carry):
        g0 = 2 * t
        _stage_in(x_hbm, table_hbm, base, g0 + 1, idx_b, rows_b, sem_b)
        _drain(out_hbm, base, g0, idx_a, rows_a, sem_a, table_hbm)

        @pl.when(t + 1 < N_CHUNKS // 2)
        def _():
            _stage_in(x_hbm, table_hbm, base, g0 + 2, idx_a, rows_a, sem_a)

        _drain(out_hbm, base, g0 + 1, idx_b, rows_b, sem_b, table_hbm)
        return carry

    lax.fori_loop(0, N_CHUNKS // 2, body, 0)


FMT_R = 1024
FMT_GRID = (NUM_EMB + FMT_R - 1) // FMT_R


def _fmt_body(t_ref, out_ref):
    row = lax.broadcasted_iota(jnp.int32, (D, D), 0)
    col = lax.broadcasted_iota(jnp.int32, (D, D), 1)
    ident = (row == col).astype(jnp.float32)
    out_ref[:, :D] = lax.dot_general(
        t_ref[...], ident,
        dimension_numbers=(((0,), (0,)), ((), ())),
        preferred_element_type=jnp.float32,
        precision=lax.Precision.HIGHEST,
    )
    out_ref[:, D:] = jnp.zeros((FMT_R, W - D), jnp.float32)


def _format_table(table):
    return pl.pallas_call(
        _fmt_body,
        out_shape=jax.ShapeDtypeStruct((NUM_EMB, W), jnp.float32),
        grid=(FMT_GRID,),
        in_specs=[pl.BlockSpec((D, FMT_R), lambda i: (0, i))],
        out_specs=pl.BlockSpec((FMT_R, W), lambda i: (i, 0)),
    )(table.T)


def _emb_lookup(x, table):
    mesh = plsc.VectorSubcoreMesh(core_axis_name="c", subcore_axis_name="s")
    x_flat = x.reshape(-1).astype(jnp.int32)
    table_pad = _format_table(table)
    out_p = pl.kernel(
        _emb_body,
        out_type=jax.ShapeDtypeStruct((B_TOTAL, W), jnp.float32),
        mesh=mesh,
        scratch_types=[
            pltpu.VMEM((CHUNK,), jnp.int32),
            pltpu.VMEM((CHUNK, W), jnp.float32),
            pltpu.SemaphoreType.DMA,
            pltpu.VMEM((CHUNK,), jnp.int32),
            pltpu.VMEM((CHUNK, W), jnp.float32),
            pltpu.SemaphoreType.DMA,
        ],
        compiler_params=pltpu.CompilerParams(use_tc_tiling_on_sc=False),
    )(x_flat, table_pad)
    return out_p.reshape(x.shape + (W,))[:, :, :D]


def kernel(x, embedding_weight):
    return _emb_lookup(x, embedding_weight)


# speedup vs baseline: 1.6723x; 1.3475x over previous
import jax
import jax.numpy as jnp
from jax import lax
from jax.experimental import pallas as pl
from jax.experimental.pallas import tpu as pltpu
from jax.experimental.pallas import tpu_sc as plsc

NUM_CORES = 2
NUM_SUBCORES = 16
NUM_WORKERS = NUM_CORES * NUM_SUBCORES

NUM_EMB = 1000000
B_TOTAL = 4096 * 200
D = 64
W = 2 * D
B_PER_W = B_TOTAL // NUM_WORKERS
CHUNK = 400
N_CHUNKS = B_PER_W // CHUNK


def _stage_in(x_hbm, table_hbm, base, g, idx_v, rows_v, sem):
    off = base + g * CHUNK
    pltpu.sync_copy(x_hbm.at[pl.ds(off, CHUNK)], idx_v)
    return pltpu.async_copy(table_hbm.at[idx_v], rows_v, sem)


def _drain(out_hbm, base, g, idx_v, rows_v, sem, table_hbm):
    off = base + g * CHUNK
    pltpu.make_async_copy(table_hbm.at[idx_v], rows_v, sem).wait()
    pltpu.sync_copy(rows_v, out_hbm.at[pl.ds(off, CHUNK)])


def _emb_body(x_hbm, table_hbm, out_hbm,
              idx_a, rows_a, sem_a, idx_b, rows_b, sem_b):
    wid = lax.axis_index("s") * NUM_CORES + lax.axis_index("c")
    base = wid * B_PER_W

    _stage_in(x_hbm, table_hbm, base, 0, idx_a, rows_a, sem_a)

    def body(t, carry):
        g0 = 2 * t
        _stage_in(x_hbm, table_hbm, base, g0 + 1, idx_b, rows_b, sem_b)
        _drain(out_hbm, base, g0, idx_a, rows_a, sem_a, table_hbm)

        @pl.when(t + 1 < N_CHUNKS // 2)
        def _():
            _stage_in(x_hbm, table_hbm, base, g0 + 2, idx_a, rows_a, sem_a)

        _drain(out_hbm, base, g0 + 1, idx_b, rows_b, sem_b, table_hbm)
        return carry

    lax.fori_loop(0, N_CHUNKS // 2, body, 0)


def _emb_lookup(x, table):
    mesh = plsc.VectorSubcoreMesh(core_axis_name="c", subcore_axis_name="s")
    x_flat = x.reshape(-1).astype(jnp.int32)
    table_pad = jnp.pad(table, ((0, 0), (0, W - D)))
    out_p = pl.kernel(
        _emb_body,
        out_type=jax.ShapeDtypeStruct((B_TOTAL, W), jnp.float32),
        mesh=mesh,
        scratch_types=[
            pltpu.VMEM((CHUNK,), jnp.int32),
            pltpu.VMEM((CHUNK, W), jnp.float32),
            pltpu.SemaphoreType.DMA,
            pltpu.VMEM((CHUNK,), jnp.int32),
            pltpu.VMEM((CHUNK, W), jnp.float32),
            pltpu.SemaphoreType.DMA,
        ],
        compiler_params=pltpu.CompilerParams(use_tc_tiling_on_sc=False),
    )(x_flat, table_pad)
    return out_p.reshape(x.shape + (W,))[:, :, :D]


def kernel(x, embedding_weight):
    return _emb_lookup(x, embedding_weight)
